# Initial kernel scaffold; baseline (speedup 1.0000x reference)
import jax
import jax.numpy as jnp
from jax.experimental import pallas as pl


def kernel(x, edge_index, edge_weights, W1, b1, We1, be1, W2, b2, We2, be2):
    raise NotImplementedError("write your pallas kernel here")


# speedup vs baseline: 4.4513x; 4.4513x over previous
import functools

import jax
import jax.numpy as jnp
from jax import lax
from jax.experimental import pallas as pl
from jax.experimental.pallas import tpu as pltpu
from jax.experimental.pallas import tpu_sc as plsc

N = 10000
D = 128
E = 320000
NC = 2
NS = 16
NW = NC * NS
EPW = E // NW
CHUNK = 128
FULL_CHUNKS = EPW // CHUNK
TAIL = EPW - FULL_CHUNKS * CHUNK
NPAD = 10112
RCH = 128
NRCH = NPAD // RCH


def _agg_body(h_hbm, src_hbm, dst_hbm, ew_hbm, we_hbm, be_hbm, z_hbm, out_hbm,
              sidx, didx, ewv, rows, tsidx, tdidx, tewv, trows,
              wev, bev, obuf, agg, sem):
    c = lax.axis_index("c")
    s = lax.axis_index("s")
    wid = s * NC + c
    base = wid * EPW

    pltpu.sync_copy(we_hbm, wev)
    pltpu.sync_copy(be_hbm, bev)
    we_r = [wev[pl.ds(16 * j, 16)] for j in range(8)]
    be_r = [bev[pl.ds(16 * j, 16)] for j in range(8)]

    pltpu.sync_copy(z_hbm, obuf)
    for k in range(5):
        idx = s + k * NS
        @pl.when(idx < NRCH)
        def _():
            pltpu.sync_copy(obuf, agg.at[pl.ds(idx * RCH, RCH)])
    plsc.subcore_barrier()

    def do_chunk(cb, n_edges, sidx_b, didx_b, ewv_b, rows_b):
        pltpu.sync_copy(src_hbm.at[pl.ds(cb, n_edges)], sidx_b)
        pltpu.sync_copy(dst_hbm.at[pl.ds(cb, n_edges)], didx_b)
        pltpu.sync_copy(ew_hbm.at[pl.ds(cb, n_edges)], ewv_b)
        pltpu.async_copy(h_hbm.at[sidx_b], rows_b, sem).wait()

        def group(g, carry):
            ew16 = ewv_b[pl.ds(16 * g, 16)]
            for l in range(16):
                i = 16 * g + l
                ewb = jnp.full((16,), ew16[l])
                for j in range(8):
                    gvec = rows_b[i, pl.ds(16 * j, 16)]
                    m = jnp.maximum(gvec + ewb * we_r[j] + be_r[j], 0.0)
                    rows_b[i, pl.ds(16 * j, 16)] = m
            return carry

        lax.fori_loop(0, n_edges // 16, group, 0)
        pltpu.sync_copy(rows_b, agg.at[didx_b], add=True)

    def chunk_loop(ci, carry):
        do_chunk(base + ci * CHUNK, CHUNK, sidx, didx, ewv, rows)
        return carry

    lax.fori_loop(0, FULL_CHUNKS, chunk_loop, 0)
    do_chunk(base + FULL_CHUNKS * CHUNK, TAIL, tsidx, tdidx, tewv, trows)

    plsc.subcore_barrier()
    for k in range(5):
        idx = s + k * NS
        @pl.when(idx < NRCH)
        def _():
            r = idx * RCH
            pltpu.sync_copy(agg.at[pl.ds(r, RCH)], obuf)
            pltpu.sync_copy(obuf, out_hbm.at[c, pl.ds(r, RCH)])


def _sc_aggregate(h, src, dst, ew, we, be, zeros):
    mesh = plsc.VectorSubcoreMesh(core_axis_name="c", subcore_axis_name="s")
    kern = pl.kernel(
        _agg_body,
        mesh=mesh,
        out_type=jax.ShapeDtypeStruct((NC, NPAD, D), jnp.float32),
        scratch_types=[
            pltpu.VMEM((CHUNK,), jnp.int32),
            pltpu.VMEM((CHUNK,), jnp.int32),
            pltpu.VMEM((CHUNK,), jnp.float32),
            pltpu.VMEM((CHUNK, D), jnp.float32),
            pltpu.VMEM((TAIL,), jnp.int32),
            pltpu.VMEM((TAIL,), jnp.int32),
            pltpu.VMEM((TAIL,), jnp.float32),
            pltpu.VMEM((TAIL, D), jnp.float32),
            pltpu.VMEM((D,), jnp.float32),
            pltpu.VMEM((D,), jnp.float32),
            pltpu.VMEM((RCH, D), jnp.float32),
            pltpu.VMEM_SHARED((NPAD, D), jnp.float32),
            pltpu.SemaphoreType.DMA,
        ],
    )
    return kern(h, src, dst, ew, we, be, zeros)


def _update_body(apply_relu, h_ref, a0_ref, a1_ref, w_ref, b_ref, o_ref):
    hs = h_ref[...] + a0_ref[0] + a1_ref[0]
    y = lax.dot_general(hs, w_ref[...], (((1,), (1,)), ((), ())),
                        preferred_element_type=jnp.float32)
    y = y + b_ref[...]
    if apply_relu:
        y = jnp.maximum(y, 0.0)
    o_ref[...] = y


def _tc_update(h, agg, W, b, apply_relu):
    BN = 1000
    nblk = N // BN
    return pl.pallas_call(
        functools.partial(_update_body, apply_relu),
        grid=(nblk,),
        in_specs=[
            pl.BlockSpec((BN, D), lambda i: (i, 0)),
            pl.BlockSpec((1, BN, D), lambda i: (0, i, 0)),
            pl.BlockSpec((1, BN, D), lambda i: (1, i, 0)),
            pl.BlockSpec((D, D), lambda i: (0, 0)),
            pl.BlockSpec((1, D), lambda i: (0, 0)),
        ],
        out_specs=pl.BlockSpec((BN, D), lambda i: (i, 0)),
        out_shape=jax.ShapeDtypeStruct((N, D), jnp.float32),
    )(h, agg, agg, W, b.reshape(1, D))


def kernel(x, edge_index, edge_weights, W1, b1, We1, be1, W2, b2, We2, be2):
    src = edge_index[0].astype(jnp.int32)
    dst = edge_index[1].astype(jnp.int32)
    ew = edge_weights.reshape(E).astype(jnp.float32)
    zeros = jnp.zeros((RCH, D), jnp.float32)
    agg1 = _sc_aggregate(x, src, dst, ew, We1[:, 0], be1, zeros)
    h2 = _tc_update(x, agg1, W1, b1, True)
    agg2 = _sc_aggregate(h2, src, dst, ew, We2[:, 0], be2, zeros)
    return _tc_update(h2, agg2, W2, b2, False)


# speedup vs baseline: 8.5192x; 1.9139x over previous
import functools

import jax
import jax.numpy as jnp
from jax import lax
from jax.experimental import pallas as pl
from jax.experimental.pallas import tpu as pltpu
from jax.experimental.pallas import tpu_sc as plsc

N = 10000
D = 128
E = 320000
NC = 2
NS = 16
NW = NC * NS
CHUNK = 128
CPW = 80
NCHUNKS = CPW * NW
EPAD = NCHUNKS * CHUNK
NPAD = 10008
RCH = 128
NFCH = NPAD // RCH
TAILR = NPAD - NFCH * RCH


def _agg_body(h_hbm, src_hbm, dst_hbm, ew_hbm, we_hbm, be_hbm, z_hbm, out_hbm,
              r0, r1, r2, sidx, didx, ewr, wev, bev, agg,
              g0, g1, g2, s0, s1, s2, i0, i1, i2):
    c = lax.axis_index("c")
    s = lax.axis_index("s")
    wid = s * NC + c
    ebase = wid * CPW * CHUNK
    rows = [r0, r1, r2]
    gsem = [g0, g1, g2]
    ssem = [s0, s1, s2]
    isem = [i0, i1, i2]

    pltpu.sync_copy(we_hbm, wev)
    pltpu.sync_copy(be_hbm, bev)
    we_r = [wev[pl.ds(16 * j, 16)] for j in range(8)]
    be_r = [bev[pl.ds(16 * j, 16)] for j in range(8)]

    pltpu.sync_copy(z_hbm, r0)
    for k in range(5):
        idx = s + k * NS
        @pl.when(idx < NFCH)
        def _():
            pltpu.sync_copy(r0, agg.at[pl.ds(idx * RCH, RCH)])
    @pl.when(s == NS - 1)
    def _():
        pltpu.sync_copy(r0.at[pl.ds(0, TAILR)],
                        agg.at[pl.ds(NFCH * RCH, TAILR)])
    plsc.subcore_barrier()

    def issue_idx(t, b):
        eb = ebase + t * CHUNK
        pltpu.async_copy(src_hbm.at[pl.ds(eb, CHUNK)], sidx.at[b], isem[b])
        pltpu.async_copy(dst_hbm.at[pl.ds(eb, CHUNK)], didx.at[b], isem[b])
        pltpu.async_copy(ew_hbm.at[pl.ds(eb, CHUNK)], ewr.at[b], isem[b])

    def wait_idx(t, b):
        eb = ebase + t * CHUNK
        pltpu.make_async_copy(src_hbm.at[pl.ds(eb, CHUNK)], sidx.at[b],
                              isem[b]).wait()
        pltpu.make_async_copy(dst_hbm.at[pl.ds(eb, CHUNK)], didx.at[b],
                              isem[b]).wait()
        pltpu.make_async_copy(ew_hbm.at[pl.ds(eb, CHUNK)], ewr.at[b],
                              isem[b]).wait()

    def issue_gather(b):
        pltpu.async_copy(h_hbm.at[sidx.at[b]], rows[b], gsem[b])

    def wait_gather(b):
        pltpu.make_async_copy(h_hbm.at[sidx.at[b]], rows[b], gsem[b]).wait()

    def issue_scatter(b):
        pltpu.async_copy(rows[b], agg.at[didx.at[b]], ssem[b], add=True)

    def wait_scatter(b):
        pltpu.make_async_copy(rows[b], agg.at[didx.at[b]], ssem[b]).wait()

    def compute(b):
        rows_b = rows[b]

        def group(g, carry):
            ew16 = ewr[b, pl.ds(16 * g, 16)]
            for l in range(16):
                i = 16 * g + l
                ewb = jnp.full((16,), ew16[l])
                for j in range(8):
                    gvec = rows_b[i, pl.ds(16 * j, 16)]
                    m = jnp.maximum(gvec + ewb * we_r[j] + be_r[j], 0.0)
                    rows_b[i, pl.ds(16 * j, 16)] = m
            return carry

        lax.fori_loop(0, CHUNK // 16, group, 0)

    def pipeline_step(p, carry):
        for u in range(3):
            t = 3 * p + u

            @pl.when(jnp.logical_and(t >= 3, t < CPW + 3))
            def _():
                wait_scatter(u)

            @pl.when(t < CPW)
            def _():
                issue_idx(t, u)

            bg = (u + 2) % 3

            @pl.when(jnp.logical_and(t >= 1, t <= CPW))
            def _():
                wait_idx(t - 1, bg)
                issue_gather(bg)

            bc = (u + 1) % 3

            @pl.when(jnp.logical_and(t >= 2, t <= CPW + 1))
            def _():
                wait_gather(bc)
                compute(bc)
                issue_scatter(bc)
        return carry

    lax.fori_loop(0, (CPW + 4) // 3, pipeline_step, 0)

    plsc.subcore_barrier()
    for k in range(5):
        idx = s + k * NS
        @pl.when(idx < NFCH)
        def _():
            r = idx * RCH
            pltpu.sync_copy(agg.at[pl.ds(r, RCH)], r0)
            pltpu.sync_copy(r0, out_hbm.at[c, pl.ds(r, RCH)])
    @pl.when(s == NS - 1)
    def _():
        pltpu.sync_copy(agg.at[pl.ds(NFCH * RCH, TAILR)],
                        r0.at[pl.ds(0, TAILR)])
        pltpu.sync_copy(r0.at[pl.ds(0, TAILR)],
                        out_hbm.at[c, pl.ds(NFCH * RCH, TAILR)])


def _sc_aggregate(h, src, dst, ew, we, be, zeros):
    mesh = plsc.VectorSubcoreMesh(core_axis_name="c", subcore_axis_name="s")
    kern = pl.kernel(
        _agg_body,
        mesh=mesh,
        out_type=jax.ShapeDtypeStruct((NC, NPAD, D), jnp.float32),
        scratch_types=[
            pltpu.VMEM((CHUNK, D), jnp.float32),
            pltpu.VMEM((CHUNK, D), jnp.float32),
            pltpu.VMEM((CHUNK, D), jnp.float32),
            pltpu.VMEM((3, CHUNK), jnp.int32),
            pltpu.VMEM((3, CHUNK), jnp.int32),
            pltpu.VMEM((3, CHUNK), jnp.float32),
            pltpu.VMEM((D,), jnp.float32),
            pltpu.VMEM((D,), jnp.float32),
            pltpu.VMEM_SHARED((NPAD, D), jnp.float32),
            pltpu.SemaphoreType.DMA,
            pltpu.SemaphoreType.DMA,
            pltpu.SemaphoreType.DMA,
            pltpu.SemaphoreType.DMA,
            pltpu.SemaphoreType.DMA,
            pltpu.SemaphoreType.DMA,
            pltpu.SemaphoreType.DMA,
            pltpu.SemaphoreType.DMA,
            pltpu.SemaphoreType.DMA,
        ],
    )
    return kern(h, src, dst, ew, we, be, zeros)


def _update_body(apply_relu, h_ref, a0_ref, a1_ref, w_ref, b_ref, o_ref):
    hs = h_ref[...] + a0_ref[0] + a1_ref[0]
    y = lax.dot_general(hs, w_ref[...], (((1,), (1,)), ((), ())),
                        preferred_element_type=jnp.float32)
    y = y + b_ref[...]
    if apply_relu:
        y = jnp.maximum(y, 0.0)
    o_ref[...] = y


def _tc_update(h, agg, W, b, apply_relu):
    BN = 1000
    nblk = N // BN
    return pl.pallas_call(
        functools.partial(_update_body, apply_relu),
        grid=(nblk,),
        in_specs=[
            pl.BlockSpec((BN, D), lambda i: (i, 0)),
            pl.BlockSpec((1, BN, D), lambda i: (0, i, 0)),
            pl.BlockSpec((1, BN, D), lambda i: (1, i, 0)),
            pl.BlockSpec((D, D), lambda i: (0, 0)),
            pl.BlockSpec((1, D), lambda i: (0, 0)),
        ],
        out_specs=pl.BlockSpec((BN, D), lambda i: (i, 0)),
        out_shape=jax.ShapeDtypeStruct((N, D), jnp.float32),
    )(h, agg, agg, W, b.reshape(1, D))


def kernel(x, edge_index, edge_weights, W1, b1, We1, be1, W2, b2, We2, be2):
    pad = EPAD - E
    pidx = jnp.arange(pad, dtype=jnp.int32)
    src = jnp.concatenate([edge_index[0].astype(jnp.int32), pidx % N])
    dst = jnp.concatenate(
        [edge_index[1].astype(jnp.int32), N + pidx % (NPAD - N)])
    ew = jnp.concatenate(
        [edge_weights.astype(jnp.float32).reshape(E),
         jnp.zeros((pad,), jnp.float32)])
    zeros = jnp.zeros((RCH, D), jnp.float32)
    agg1 = _sc_aggregate(x, src, dst, ew, We1[:, 0], be1, zeros)
    h2 = _tc_update(x, agg1, W1, b1, True)
    agg2 = _sc_aggregate(h2, src, dst, ew, We2[:, 0], be2, zeros)
    return _tc_update(h2, agg2, W2, b2, False)


# speedup vs baseline: 9.6874x; 1.1371x over previous
import functools

import jax
import jax.numpy as jnp
from jax import lax
from jax.experimental import pallas as pl
from jax.experimental.pallas import tpu as pltpu
from jax.experimental.pallas import tpu_sc as plsc

N = 10000
D = 128
E = 320000
NC = 2
NS = 16
NW = NC * NS
CHUNK = 128
CPW = 80
NCHUNKS = CPW * NW
EPAD = NCHUNKS * CHUNK
NPAD = 10008
RCH = 128
NFCH = NPAD // RCH
TAILR = NPAD - NFCH * RCH


def _agg_body(h_hbm, ed_hbm, ew_hbm, we_hbm, z_hbm, out_hbm,
              r0, r1, r2, er, ewr, wev, agg,
              g0, g1, g2, s0, s1, s2, i0, i1, i2):
    c = lax.axis_index("c")
    s = lax.axis_index("s")
    wid = s * NC + c
    cbase = wid * CPW
    rows = [r0, r1, r2]
    gsem = [g0, g1, g2]
    ssem = [s0, s1, s2]
    isem = [i0, i1, i2]

    pltpu.sync_copy(we_hbm, wev)
    we_r = [wev[pl.ds(16 * j, 16)] for j in range(8)]

    pltpu.sync_copy(z_hbm, r0)
    for k in range(5):
        idx = s + k * NS
        @pl.when(idx < NFCH)
        def _():
            pltpu.sync_copy(r0, agg.at[pl.ds(idx * RCH, RCH)])
    @pl.when(s == NS - 1)
    def _():
        pltpu.sync_copy(r0.at[pl.ds(0, TAILR)],
                        agg.at[pl.ds(NFCH * RCH, TAILR)])
    plsc.subcore_barrier()

    def issue_idx(t, b):
        pltpu.async_copy(ed_hbm.at[cbase + t], er.at[b], isem[b])
        pltpu.async_copy(ew_hbm.at[pl.ds((cbase + t) * CHUNK, CHUNK)],
                         ewr.at[b], isem[b])

    def wait_idx(t, b):
        pltpu.make_async_copy(ed_hbm.at[cbase + t], er.at[b], isem[b]).wait()
        pltpu.make_async_copy(ew_hbm.at[pl.ds((cbase + t) * CHUNK, CHUNK)],
                              ewr.at[b], isem[b]).wait()

    def issue_gather(b):
        pltpu.async_copy(h_hbm.at[er.at[b, 0]], rows[b], gsem[b])

    def wait_gather(b):
        pltpu.make_async_copy(h_hbm.at[er.at[b, 0]], rows[b], gsem[b]).wait()

    def issue_scatter(b):
        pltpu.async_copy(rows[b], agg.at[er.at[b, 1]], ssem[b], add=True)

    def wait_scatter(b):
        pltpu.make_async_copy(rows[b], agg.at[er.at[b, 1]], ssem[b]).wait()

    def compute(b):
        rows_b = rows[b]

        @plsc.parallel_loop(0, CHUNK // 16, unroll=2)
        def _(g):
            ew16 = ewr[b, pl.ds(16 * g, 16)]
            for l in range(16):
                i = 16 * g + l
                ewb = jnp.full((16,), ew16[l])
                for j in range(8):
                    gvec = rows_b[i, pl.ds(16 * j, 16)]
                    m = jnp.maximum(gvec + ewb * we_r[j], 0.0)
                    rows_b[i, pl.ds(16 * j, 16)] = m

    def pipeline_step(p, carry):
        for u in range(3):
            t = 3 * p + u

            @pl.when(jnp.logical_and(t >= 3, t < CPW + 3))
            def _():
                wait_scatter(u)

            @pl.when(t < CPW)
            def _():
                issue_idx(t, u)

            bg = (u + 2) % 3

            @pl.when(jnp.logical_and(t >= 1, t <= CPW))
            def _():
                wait_idx(t - 1, bg)
                issue_gather(bg)

            bc = (u + 1) % 3

            @pl.when(jnp.logical_and(t >= 2, t <= CPW + 1))
            def _():
                wait_gather(bc)
                compute(bc)
                issue_scatter(bc)
        return carry

    lax.fori_loop(0, (CPW + 4) // 3, pipeline_step, 0)

    plsc.subcore_barrier()
    for k in range(5):
        idx = s + k * NS
        @pl.when(idx < NFCH)
        def _():
            r = idx * RCH
            pltpu.sync_copy(agg.at[pl.ds(r, RCH)], r0)
            pltpu.sync_copy(r0, out_hbm.at[c, pl.ds(r, RCH)])
    @pl.when(s == NS - 1)
    def _():
        pltpu.sync_copy(agg.at[pl.ds(NFCH * RCH, TAILR)],
                        r0.at[pl.ds(0, TAILR)])
        pltpu.sync_copy(r0.at[pl.ds(0, TAILR)],
                        out_hbm.at[c, pl.ds(NFCH * RCH, TAILR)])


def _sc_aggregate(h, edata, ew, we, zeros):
    mesh = plsc.VectorSubcoreMesh(core_axis_name="c", subcore_axis_name="s")
    kern = pl.kernel(
        _agg_body,
        mesh=mesh,
        out_type=jax.ShapeDtypeStruct((NC, NPAD, D), jnp.float32),
        scratch_types=[
            pltpu.VMEM((CHUNK, D), jnp.float32),
            pltpu.VMEM((CHUNK, D), jnp.float32),
            pltpu.VMEM((CHUNK, D), jnp.float32),
            pltpu.VMEM((3, 2, CHUNK), jnp.int32),
            pltpu.VMEM((3, CHUNK), jnp.float32),
            pltpu.VMEM((D,), jnp.float32),
            pltpu.VMEM_SHARED((NPAD, D), jnp.float32),
            pltpu.SemaphoreType.DMA,
            pltpu.SemaphoreType.DMA,
            pltpu.SemaphoreType.DMA,
            pltpu.SemaphoreType.DMA,
            pltpu.SemaphoreType.DMA,
            pltpu.SemaphoreType.DMA,
            pltpu.SemaphoreType.DMA,
            pltpu.SemaphoreType.DMA,
            pltpu.SemaphoreType.DMA,
        ],
    )
    return kern(h, edata, ew, we, zeros)


def _bias_body(x_ref, b_ref, o_ref):
    o_ref[...] = x_ref[...] + b_ref[...]


def _bias_add(x, be):
    BN = 2000
    return pl.pallas_call(
        _bias_body,
        grid=(N // BN,),
        in_specs=[
            pl.BlockSpec((BN, D), lambda i: (i, 0)),
            pl.BlockSpec((1, D), lambda i: (0, 0)),
        ],
        out_specs=pl.BlockSpec((BN, D), lambda i: (i, 0)),
        out_shape=jax.ShapeDtypeStruct((N, D), jnp.float32),
    )(x, be.reshape(1, D))


def _update_body(apply_relu, h_ref, a0_ref, a1_ref, w_ref, b_ref, be2_ref,
                 o_ref, ob_ref=None):
    hs = h_ref[...] + a0_ref[0] + a1_ref[0]
    y = lax.dot_general(hs, w_ref[...], (((1,), (1,)), ((), ())),
                        preferred_element_type=jnp.float32)
    y = y + b_ref[...]
    if apply_relu:
        y = jnp.maximum(y, 0.0)
    o_ref[...] = y
    if ob_ref is not None:
        ob_ref[...] = y + be2_ref[...]


def _tc_update(h, agg, W, b, be_next, apply_relu):
    BN = 1000
    nblk = N // BN
    two_out = be_next is not None
    ospec = pl.BlockSpec((BN, D), lambda i: (i, 0))
    oshape = jax.ShapeDtypeStruct((N, D), jnp.float32)
    if be_next is None:
        be_next = b
    return pl.pallas_call(
        functools.partial(_update_body, apply_relu),
        grid=(nblk,),
        in_specs=[
            pl.BlockSpec((BN, D), lambda i: (i, 0)),
            pl.BlockSpec((1, BN, D), lambda i: (0, i, 0)),
            pl.BlockSpec((1, BN, D), lambda i: (1, i, 0)),
            pl.BlockSpec((D, D), lambda i: (0, 0)),
            pl.BlockSpec((1, D), lambda i: (0, 0)),
            pl.BlockSpec((1, D), lambda i: (0, 0)),
        ],
        out_specs=[ospec, ospec] if two_out else [ospec],
        out_shape=[oshape, oshape] if two_out else [oshape],
    )(h, agg, agg, W, b.reshape(1, D), be_next.reshape(1, D))


def kernel(x, edge_index, edge_weights, W1, b1, We1, be1, W2, b2, We2, be2):
    pad = EPAD - E
    pidx = jnp.arange(pad, dtype=jnp.int32)
    src = jnp.concatenate([edge_index[0].astype(jnp.int32), pidx % N])
    dst = jnp.concatenate(
        [edge_index[1].astype(jnp.int32), N + pidx % (NPAD - N)])
    ew = jnp.concatenate(
        [edge_weights.astype(jnp.float32).reshape(E),
         jnp.zeros((pad,), jnp.float32)])
    edata = jnp.stack(
        [src.reshape(NCHUNKS, CHUNK), dst.reshape(NCHUNKS, CHUNK)], axis=1)
    zeros = jnp.zeros((RCH, D), jnp.float32)
    hb1 = _bias_add(x, be1)
    agg1 = _sc_aggregate(hb1, edata, ew, We1[:, 0], zeros)
    h2, hb2 = _tc_update(x, agg1, W1, b1, be2, True)
    agg2 = _sc_aggregate(hb2, edata, ew, We2[:, 0], zeros)
    (out,) = _tc_update(h2, agg2, W2, b2, None, False)
    return out
